# baseline (speedup 1.0000x reference)
import functools

import jax
import jax.numpy as jnp
from jax import lax
from jax.experimental import pallas as pl
from jax.experimental.pallas import tpu as pltpu
from jax.experimental.pallas import tpu_sc as plsc

NC, NS = 2, 16
NW = NC * NS
BB = 128
L = 16


def kernel(xs, weight):
    B, H = xs.shape
    V, D = weight.shape
    n_units = H
    assert B == NW * BB and D == 32 and V % 4 == 0

    xs_t = jnp.transpose(xs).astype(jnp.int32)
    w128 = jnp.reshape(weight, (V // 4, 4 * D))
    mesh = plsc.VectorSubcoreMesh(core_axis_name="c", subcore_axis_name="s")

    @functools.partial(
        pl.kernel,
        out_type=jax.ShapeDtypeStruct((H, D, B), jnp.float32),
        mesh=mesh,
        scratch_types=[
            pltpu.VMEM((H, BB), jnp.int32),
            pltpu.VMEM((2, BB, 4 * D), jnp.float32),
            pltpu.VMEM((2, D, BB), jnp.float32),
            pltpu.VMEM((2, BB), jnp.int32),
            pltpu.SemaphoreType.DMA,
            pltpu.SemaphoreType.DMA,
            pltpu.SemaphoreType.DMA,
            pltpu.SemaphoreType.DMA,
        ],
        compiler_params=pltpu.CompilerParams(
            use_tc_tiling_on_sc=True, needs_layout_passes=False),
    )
    def run(xs_hbm, w_hbm, out_hbm, xsb_v, rows_v, stage_v, idxq_v,
            gsem0, gsem1, osem0, osem1):
        wid = lax.axis_index("s") * NC + lax.axis_index("c")
        b0 = wid * BB

        gsems = (gsem0, gsem1)
        osems = (osem0, osem1)
        iota = lax.iota(jnp.int32, L)

        pltpu.sync_copy(xs_hbm.at[pl.ds(0, H), pl.ds(b0, BB)], xsb_v)

        def prep_fill(slot, u):
            for j in range(BB // L):
                bidx = xsb_v[u, pl.ds(j * L, L)]
                idxq_v[slot, pl.ds(j * L, L)] = lax.shift_right_logical(bidx, 2)
            pltpu.async_copy(
                w_hbm.at[idxq_v.at[slot]], rows_v.at[slot], gsems[slot]
            )

        def gwait(slot):
            pltpu.make_async_copy(
                w_hbm.at[pl.ds(0, BB)], rows_v.at[slot], gsems[slot]
            ).wait()

        def transpose(slot, u):
            for j in range(BB // L):
                bidx = xsb_v[u, pl.ds(j * L, L)]
                cbase = (bidx & 3) * 32
                rowv = iota + (j * L)
                for d in range(D):
                    vec = plsc.load_gather(
                        rows_v.at[slot], [rowv, cbase + d]
                    )
                    stage_v[slot, d, pl.ds(j * L, L)] = vec

        def ostart(slot, u):
            for td in range(4):
                pltpu.async_copy(
                    stage_v.at[slot].at[pl.ds(td * 8, 8)],
                    out_hbm.at[u].at[pl.ds(td * 8, 8), pl.ds(b0, BB)],
                    osems[slot],
                )

        def owait(slot):
            pltpu.make_async_copy(
                stage_v.at[slot],
                out_hbm.at[0].at[pl.ds(0, D), pl.ds(b0, BB)],
                osems[slot],
            ).wait()

        prep_fill(0, 0)
        prep_fill(1, 1)

        @pl.loop(0, n_units, step=2)
        def _(u):
            for slot in (0, 1):
                uu = u + slot
                gwait(slot)

                @pl.when(uu >= 2)
                def _():
                    owait(slot)

                transpose(slot, uu)

                @pl.when(uu + 2 < n_units)
                def _():
                    prep_fill(slot, uu + 2)

                ostart(slot, uu)

        owait(0)
        owait(1)

    out = run(xs_t, w128)
    return jnp.transpose(out, (2, 0, 1))


# speedup vs baseline: 1.3943x; 1.3943x over previous
import functools

import jax
import jax.numpy as jnp
from jax import lax
from jax.experimental import pallas as pl
from jax.experimental.pallas import tpu as pltpu
from jax.experimental.pallas import tpu_sc as plsc

NC, NS = 2, 16
NW = NC * NS
BB = 128
L = 16
NBUF = 3


def kernel(xs, weight):
    B, H = xs.shape
    V, D = weight.shape
    n_units = H
    assert B == NW * BB and D == 32 and V % 4 == 0

    xs_t = jnp.transpose(xs).astype(jnp.int32)
    w128 = jnp.reshape(weight, (V // 4, 4 * D))
    mesh = plsc.VectorSubcoreMesh(core_axis_name="c", subcore_axis_name="s")

    @functools.partial(
        pl.kernel,
        out_type=jax.ShapeDtypeStruct((H, D, B), jnp.float32),
        mesh=mesh,
        scratch_types=[
            pltpu.VMEM((H, BB), jnp.int32),
            pltpu.VMEM((NBUF, BB, 4 * D), jnp.float32),
            pltpu.VMEM((NBUF, D, BB), jnp.float32),
            pltpu.VMEM((NBUF, BB), jnp.int32),
            pltpu.SemaphoreType.DMA,
            pltpu.SemaphoreType.DMA,
            pltpu.SemaphoreType.DMA,
            pltpu.SemaphoreType.DMA,
            pltpu.SemaphoreType.DMA,
            pltpu.SemaphoreType.DMA,
        ],
        compiler_params=pltpu.CompilerParams(
            use_tc_tiling_on_sc=True, needs_layout_passes=False),
    )
    def run(xs_hbm, w_hbm, out_hbm, xsb_v, rows_v, stage_v, idxq_v,
            gsem0, gsem1, gsem2, osem0, osem1, osem2):
        wid = lax.axis_index("s") * NC + lax.axis_index("c")
        b0 = wid * BB

        gsems = (gsem0, gsem1, gsem2)
        osems = (osem0, osem1, osem2)
        iota = lax.iota(jnp.int32, L)

        pltpu.sync_copy(xs_hbm.at[pl.ds(0, H), pl.ds(b0, BB)], xsb_v)

        def prep_fill(slot, u):
            for j in range(BB // L):
                bidx = xsb_v[u, pl.ds(j * L, L)]
                idxq_v[slot, pl.ds(j * L, L)] = lax.shift_right_logical(bidx, 2)
            pltpu.async_copy(
                w_hbm.at[idxq_v.at[slot]], rows_v.at[slot], gsems[slot]
            )

        def gwait(slot):
            pltpu.make_async_copy(
                w_hbm.at[pl.ds(0, BB)], rows_v.at[slot], gsems[slot]
            ).wait()

        def transpose(slot, u):
            for j in range(BB // L):
                bidx = xsb_v[u, pl.ds(j * L, L)]
                cbase = (bidx & 3) * 32
                rowv = iota + (j * L)

                @pl.loop(0, D, unroll=8)
                def _(d0):
                    dvec = (iota + d0) & 31
                    vec = plsc.load_gather(rows_v.at[slot], [rowv, cbase + dvec])
                    plsc.store_scatter(stage_v.at[slot], [dvec, rowv], vec)

        def ostart(slot, u):
            for td in range(4):
                pltpu.async_copy(
                    stage_v.at[slot].at[pl.ds(td * 8, 8)],
                    out_hbm.at[u].at[pl.ds(td * 8, 8), pl.ds(b0, BB)],
                    osems[slot],
                )

        def owait(slot):
            pltpu.make_async_copy(
                stage_v.at[slot],
                out_hbm.at[0].at[pl.ds(0, D), pl.ds(b0, BB)],
                osems[slot],
            ).wait()

        for s in range(NBUF - 1):
            prep_fill(s, s)

        @pl.loop(0, n_units, step=NBUF)
        def _(u):
            for k in range(NBUF):
                uu = u + k

                @pl.when(uu < n_units)
                def _():
                    gwait(k)

                    @pl.when(uu >= NBUF)
                    def _():
                        owait(k)

                    transpose(k, uu)

                    @pl.when(uu + 2 < n_units)
                    def _():
                        prep_fill((k + 2) % NBUF, uu + 2)

                    ostart(k, uu)

        for s in range(NBUF):
            owait(s)

    out = run(xs_t, w128)
    return jnp.transpose(out, (2, 0, 1))


# speedup vs baseline: 1.4888x; 1.0677x over previous
import functools

import jax
import jax.numpy as jnp
from jax import lax
from jax.experimental import pallas as pl
from jax.experimental.pallas import tpu as pltpu
from jax.experimental.pallas import tpu_sc as plsc

NC, NS = 2, 16
NW = NC * NS
BB = 128
L = 16
NBUF = 3


def kernel(xs, weight):
    B, H = xs.shape
    V, D = weight.shape
    n_units = H
    assert B == NW * BB and D == 32 and V % 4 == 0

    xs_t = jnp.transpose(xs).astype(jnp.int32)
    w128 = jnp.reshape(weight, (V // 4, 4 * D))
    mesh = plsc.VectorSubcoreMesh(core_axis_name="c", subcore_axis_name="s")

    @functools.partial(
        pl.kernel,
        out_type=jax.ShapeDtypeStruct((H, D, B), jnp.float32),
        mesh=mesh,
        scratch_types=[
            pltpu.VMEM((H, BB), jnp.int32),
            pltpu.VMEM((NBUF, BB, 4 * D), jnp.float32),
            pltpu.VMEM((NBUF, D, BB), jnp.float32),
            pltpu.VMEM((NBUF, BB), jnp.int32),
            pltpu.SemaphoreType.DMA,
            pltpu.SemaphoreType.DMA,
            pltpu.SemaphoreType.DMA,
            pltpu.SemaphoreType.DMA,
            pltpu.SemaphoreType.DMA,
            pltpu.SemaphoreType.DMA,
        ],
        compiler_params=pltpu.CompilerParams(
            use_tc_tiling_on_sc=True, needs_layout_passes=False),
    )
    def run(xs_hbm, w_hbm, out_hbm, xsb_v, rows_v, stage_v, idxq_v,
            gsem0, gsem1, gsem2, osem0, osem1, osem2):
        wid = lax.axis_index("s") * NC + lax.axis_index("c")
        b0 = wid * BB

        gsems = (gsem0, gsem1, gsem2)
        osems = (osem0, osem1, osem2)
        iota = lax.iota(jnp.int32, L)

        pltpu.sync_copy(xs_hbm.at[pl.ds(0, H), pl.ds(b0, BB)], xsb_v)

        def prep_fill(slot, u):
            for j in range(BB // L):
                bidx = xsb_v[u, pl.ds(j * L, L)]
                idxq_v[slot, pl.ds(j * L, L)] = lax.shift_right_logical(bidx, 2)
            pltpu.async_copy(
                w_hbm.at[idxq_v.at[slot]], rows_v.at[slot], gsems[slot]
            )

        def gwait(slot):
            pltpu.make_async_copy(
                w_hbm.at[pl.ds(0, BB)], rows_v.at[slot], gsems[slot]
            ).wait()

        def transpose(slot, u):
            for j in range(BB // L):
                bidx = xsb_v[u, pl.ds(j * L, L)]
                cbase = (bidx & 3) * 32
                rowv = iota + (j * L)

                @pl.loop(0, D, unroll=16)
                def _(d0):
                    dvec = (iota + d0) & 31
                    vec = plsc.load_gather(rows_v.at[slot], [rowv, cbase + dvec])
                    plsc.store_scatter(stage_v.at[slot], [dvec, rowv], vec)

        def ostart(slot, u):
            for td in range(4):
                pltpu.async_copy(
                    stage_v.at[slot].at[pl.ds(td * 8, 8)],
                    out_hbm.at[u].at[pl.ds(td * 8, 8), pl.ds(b0, BB)],
                    osems[slot],
                )

        def owait(slot):
            pltpu.make_async_copy(
                stage_v.at[slot],
                out_hbm.at[0].at[pl.ds(0, D), pl.ds(b0, BB)],
                osems[slot],
            ).wait()

        for s in range(NBUF - 1):
            prep_fill(s, s)

        @pl.loop(0, n_units, step=NBUF)
        def _(u):
            for k in range(NBUF):
                uu = u + k

                @pl.when(uu < n_units)
                def _():
                    gwait(k)

                    @pl.when(uu >= NBUF)
                    def _():
                        owait(k)

                    transpose(k, uu)

                    @pl.when(uu + 2 < n_units)
                    def _():
                        prep_fill((k + 2) % NBUF, uu + 2)

                    ostart(k, uu)

        for s in range(NBUF):
            owait(s)

    out = run(xs_t, w128)
    return jnp.transpose(out, (2, 0, 1))


# speedup vs baseline: 1.5305x; 1.0280x over previous
import functools

import jax
import jax.numpy as jnp
from jax import lax
from jax.experimental import pallas as pl
from jax.experimental.pallas import tpu as pltpu
from jax.experimental.pallas import tpu_sc as plsc

NC, NS = 2, 16
NW = NC * NS
BB = 128
L = 16
NBUF = 4


def kernel(xs, weight):
    B, H = xs.shape
    V, D = weight.shape
    n_units = H
    assert B == NW * BB and D == 32 and V % 4 == 0

    xs_t = jnp.transpose(xs).astype(jnp.int32)
    w128 = jnp.reshape(weight, (V // 4, 4 * D))
    mesh = plsc.VectorSubcoreMesh(core_axis_name="c", subcore_axis_name="s")

    @functools.partial(
        pl.kernel,
        out_type=jax.ShapeDtypeStruct((H, D, B), jnp.float32),
        mesh=mesh,
        scratch_types=[
            pltpu.VMEM((H, BB), jnp.int32),
            pltpu.VMEM((NBUF, BB, 4 * D), jnp.float32),
            pltpu.VMEM((NBUF, D, BB), jnp.float32),
            pltpu.VMEM((NBUF, BB), jnp.int32),
        ] + [pltpu.SemaphoreType.DMA] * (2 * NBUF),
        compiler_params=pltpu.CompilerParams(
            use_tc_tiling_on_sc=True, needs_layout_passes=False),
    )
    def run(xs_hbm, w_hbm, out_hbm, xsb_v, rows_v, stage_v, idxq_v, *sems):
        wid = lax.axis_index("s") * NC + lax.axis_index("c")
        b0 = wid * BB

        gsems = sems[:NBUF]
        osems = sems[NBUF:]
        iota = lax.iota(jnp.int32, L)

        pltpu.sync_copy(xs_hbm.at[pl.ds(0, H), pl.ds(b0, BB)], xsb_v)

        def prep_fill(slot, u):
            for j in range(BB // L):
                bidx = xsb_v[u, pl.ds(j * L, L)]
                idxq_v[slot, pl.ds(j * L, L)] = lax.shift_right_logical(bidx, 2)
            pltpu.async_copy(
                w_hbm.at[idxq_v.at[slot]], rows_v.at[slot], gsems[slot]
            )

        def gwait(slot):
            pltpu.make_async_copy(
                w_hbm.at[pl.ds(0, BB)], rows_v.at[slot], gsems[slot]
            ).wait()

        def transpose(slot, u):
            for j in range(BB // L):
                bidx = xsb_v[u, pl.ds(j * L, L)]
                cbase = (bidx & 3) * 32
                rowv = iota + (j * L)

                @pl.loop(0, D, unroll=16)
                def _(d0):
                    dvec = (iota + d0) & 31
                    vec = plsc.load_gather(rows_v.at[slot], [rowv, cbase + dvec])
                    plsc.store_scatter(stage_v.at[slot], [dvec, rowv], vec)

        def ostart(slot, u):
            for td in range(4):
                pltpu.async_copy(
                    stage_v.at[slot].at[pl.ds(td * 8, 8)],
                    out_hbm.at[u].at[pl.ds(td * 8, 8), pl.ds(b0, BB)],
                    osems[slot],
                )

        def owait(slot):
            pltpu.make_async_copy(
                stage_v.at[slot],
                out_hbm.at[0].at[pl.ds(0, D), pl.ds(b0, BB)],
                osems[slot],
            ).wait()

        for s in range(NBUF - 1):
            prep_fill(s, s)

        @pl.loop(0, n_units, step=NBUF)
        def _(u):
            for k in range(NBUF):
                uu = u + k

                @pl.when(uu < n_units)
                def _():
                    gwait(k)

                    @pl.when(uu + (NBUF - 1) < n_units)
                    def _():
                        prep_fill((k + NBUF - 1) % NBUF, uu + NBUF - 1)

                    @pl.when(uu >= NBUF)
                    def _():
                        owait(k)

                    transpose(k, uu)

                    ostart(k, uu)

        for s in range(NBUF):
            owait(s)

    out = run(xs_t, w128)
    return jnp.transpose(out, (2, 0, 1))


# speedup vs baseline: 1.5905x; 1.0392x over previous
import functools

import jax
import jax.numpy as jnp
from jax import lax
from jax.experimental import pallas as pl
from jax.experimental.pallas import tpu as pltpu
from jax.experimental.pallas import tpu_sc as plsc

NC, NS = 2, 16
NW = NC * NS
BB = 128
L = 16
NBUF = 4


def kernel(xs, weight):
    B, H = xs.shape
    V, D = weight.shape
    n_units = H
    assert B == NW * BB and D == 32 and V % 4 == 0

    xs_t = jnp.transpose(xs).astype(jnp.int32)
    w128 = jnp.reshape(weight, (V // 4, 4 * D))
    mesh = plsc.VectorSubcoreMesh(core_axis_name="c", subcore_axis_name="s")

    @functools.partial(
        pl.kernel,
        out_type=jax.ShapeDtypeStruct((H, D, B), jnp.float32),
        mesh=mesh,
        scratch_types=[
            pltpu.VMEM((H, BB), jnp.int32),
            pltpu.VMEM((NBUF, BB, 4 * D), jnp.float32),
            pltpu.VMEM((NBUF, D, BB), jnp.float32),
            pltpu.VMEM((NBUF, BB), jnp.int32),
        ] + [pltpu.SemaphoreType.DMA] * (2 * NBUF),
        compiler_params=pltpu.CompilerParams(
            use_tc_tiling_on_sc=True, needs_layout_passes=False),
    )
    def run(xs_hbm, w_hbm, out_hbm, xsb_v, rows_v, stage_v, idxq_v, *sems):
        wid = lax.axis_index("s") * NC + lax.axis_index("c")
        b0 = wid * BB

        gsems = sems[:NBUF]
        osems = sems[NBUF:]
        iota = lax.iota(jnp.int32, L)

        pltpu.sync_copy(xs_hbm.at[pl.ds(0, H), pl.ds(b0, BB)], xsb_v)

        def prep_fill(slot, u):
            for j in range(BB // L):
                bidx = xsb_v[u, pl.ds(j * L, L)]
                idxq_v[slot, pl.ds(j * L, L)] = lax.shift_right_logical(bidx, 2)
            pltpu.async_copy(
                w_hbm.at[idxq_v.at[slot]], rows_v.at[slot], gsems[slot]
            )

        def gwait(slot):
            pltpu.make_async_copy(
                w_hbm.at[pl.ds(0, BB)], rows_v.at[slot], gsems[slot]
            ).wait()

        def transpose(slot, u):
            for j in range(BB // L):
                bidx = xsb_v[u, pl.ds(j * L, L)]
                cbase = (bidx & 3) * 32
                rowv = iota + (j * L)

                @pl.loop(0, D, unroll=16)
                def _(d0):
                    dvec = iota ^ d0
                    vec = plsc.load_gather(rows_v.at[slot], [rowv, cbase + dvec])
                    plsc.store_scatter(stage_v.at[slot], [dvec, rowv], vec)

        def ostart(slot, u):
            for td in range(4):
                pltpu.async_copy(
                    stage_v.at[slot].at[pl.ds(td * 8, 8)],
                    out_hbm.at[u].at[pl.ds(td * 8, 8), pl.ds(b0, BB)],
                    osems[slot],
                )

        def owait(slot):
            pltpu.make_async_copy(
                stage_v.at[slot],
                out_hbm.at[0].at[pl.ds(0, D), pl.ds(b0, BB)],
                osems[slot],
            ).wait()

        for s in range(NBUF - 1):
            prep_fill(s, s)

        @pl.loop(0, n_units, step=NBUF)
        def _(u):
            for k in range(NBUF):
                uu = u + k

                @pl.when(uu < n_units)
                def _():
                    gwait(k)

                    @pl.when(uu + (NBUF - 1) < n_units)
                    def _():
                        prep_fill((k + NBUF - 1) % NBUF, uu + NBUF - 1)

                    @pl.when(uu >= NBUF)
                    def _():
                        owait(k)

                    transpose(k, uu)

                    ostart(k, uu)

        for s in range(NBUF):
            owait(s)

    out = run(xs_t, w128)
    return jnp.transpose(out, (2, 0, 1))
